# baseline (speedup 1.0000x reference)
import functools
import math

import jax
import jax.numpy as jnp
from jax import lax
from jax.experimental import pallas as pl
from jax.experimental.pallas import tpu as pltpu
from jax.experimental.pallas import tpu_sc as plsc

SEQ_LEN = 2048
BATCH = 64
DIM = 64
NC = 2
NS = 16
NW = NC * NS
ROWS = SEQ_LEN * BATCH
ROWS_W = ROWS // NW
SEQ_W = SEQ_LEN // NW
CHUNK_S = 2
CHUNK_R = CHUNK_S * BATCH
N_CHUNKS = SEQ_W // CHUNK_S
SCALE = math.sqrt(DIM)
LANES = 16
VPR = DIM // LANES
N_SLOTS = 4
LOOKAHEAD = 2
TC_BS = 64


def _sc_body(idx_hbm, wt_hbm, pe_hbm, out_hbm, idx_v, pe_v, bufs, gsems, osems):
    wid = lax.axis_index("s") * NC + lax.axis_index("c")
    base = wid * ROWS_W

    pltpu.sync_copy(idx_hbm.at[pl.ds(base, ROWS_W)], idx_v)
    pltpu.sync_copy(pe_hbm.at[pl.ds(wid * SEQ_W * DIM, SEQ_W * DIM)], pe_v)

    def start_gather(g):
        slot = g % N_SLOTS
        idx_slice = idx_v.at[pl.ds(g * CHUNK_R, CHUNK_R)]
        return pltpu.async_copy(wt_hbm.at[idx_slice], bufs.at[slot], gsems[slot])

    def start_out(g):
        slot = g % N_SLOTS
        return pltpu.async_copy(
            bufs.at[slot],
            out_hbm.at[pl.ds(base + g * CHUNK_R, CHUNK_R)],
            osems[slot],
        )

    gd = {}
    od = {}
    for g in range(LOOKAHEAD):
        gd[g] = start_gather(g)

    for g in range(N_CHUNKS):
        h = g + LOOKAHEAD
        if h < N_CHUNKS:
            prev = h - N_SLOTS
            if prev >= 0:
                od.pop(prev).wait()
            gd[h] = start_gather(h)

        gd.pop(g).wait()

        slot = g % N_SLOTS
        for sp in range(CHUNK_S):
            srow = g * CHUNK_S + sp
            pe_regs = [
                pe_v[pl.ds(srow * DIM + j * LANES, LANES)] for j in range(VPR)
            ]

            def row_body(r, c, pe_regs=pe_regs, sp=sp, slot=slot):
                k = sp * BATCH + r
                for j in range(VPR):
                    v = bufs[slot, k, pl.ds(j * LANES, LANES)]
                    bufs[slot, k, pl.ds(j * LANES, LANES)] = v * SCALE + pe_regs[j]
                return c

            lax.fori_loop(0, BATCH, row_body, 0, unroll=2)

        od[g] = start_out(g)

    for g in sorted(od):
        od.pop(g).wait()


@functools.cache
def _build_sc():
    mesh = plsc.VectorSubcoreMesh(
        core_axis_name="c", subcore_axis_name="s", num_cores=NC, num_subcores=NS
    )
    return pl.kernel(
        _sc_body,
        out_type=jax.ShapeDtypeStruct((ROWS, DIM), jnp.float32),
        mesh=mesh,
        scratch_types=[
            pltpu.VMEM((ROWS_W,), jnp.int32),
            pltpu.VMEM((SEQ_W * DIM,), jnp.float32),
            pltpu.VMEM((N_SLOTS, CHUNK_R, DIM), jnp.float32),
            [pltpu.SemaphoreType.DMA] * N_SLOTS,
            [pltpu.SemaphoreType.DMA] * N_SLOTS,
        ],
        compiler_params=pltpu.CompilerParams(use_tc_tiling_on_sc=False),
    )


def _tc_body(x_ref, o_ref):
    x = x_ref[...]
    o_ref[...] = jnp.concatenate([x[:, :, :DIM], x[:, :, DIM:]], axis=1)


@functools.cache
def _build_tc():
    return pl.pallas_call(
        _tc_body,
        grid=(SEQ_LEN // TC_BS,),
        in_specs=[pl.BlockSpec((TC_BS, BATCH // 2, 2 * DIM), lambda i: (i, 0, 0))],
        out_specs=pl.BlockSpec((TC_BS, BATCH, DIM), lambda i: (i, 0, 0)),
        out_shape=jax.ShapeDtypeStruct((SEQ_LEN, BATCH, DIM), jnp.float32),
    )


def kernel(source, word_table, pe):
    idx = source.reshape(SEQ_LEN, 2, BATCH // 2).transpose(0, 2, 1).reshape(ROWS)
    pe_flat = pe[:SEQ_LEN, 0, :].reshape(SEQ_LEN * DIM)
    packed = _build_sc()(idx, word_table, pe_flat)
    packed3 = packed.reshape(SEQ_LEN, BATCH // 2, 2 * DIM)
    return _build_tc()(packed3)


# speedup vs baseline: 1.1992x; 1.1992x over previous
import functools
import math

import jax
import jax.numpy as jnp
from jax import lax
from jax.experimental import pallas as pl
from jax.experimental.pallas import tpu as pltpu
from jax.experimental.pallas import tpu_sc as plsc

SEQ_LEN = 2048
BATCH = 64
DIM = 64
NC = 2
NS = 16
NW = NC * NS
ROWS = SEQ_LEN * BATCH
ROWS_W = ROWS // NW
SEQ_W = SEQ_LEN // NW
SCALE = math.sqrt(DIM)
LANES = 16
VPR = DIM // LANES
N_SLOTS = 4
LOOKAHEAD = 2


def _sc_body(idx_hbm, wt_hbm, pe_hbm, out_hbm, idx_v, pe_v, bufs, gsems, osems):
    wid = lax.axis_index("s") * NC + lax.axis_index("c")
    base = wid * ROWS_W
    seq_base = wid * SEQ_W

    pltpu.sync_copy(idx_hbm.at[pl.ds(base, ROWS_W)], idx_v)
    pltpu.sync_copy(pe_hbm.at[pl.ds(seq_base * DIM, SEQ_W * DIM)], pe_v)

    def start_gather(s):
        slot = s % N_SLOTS
        return pltpu.async_copy(
            wt_hbm.at[idx_v.at[pl.ds(s * BATCH, BATCH)]], bufs.at[slot], gsems[slot]
        )

    def start_out(s):
        slot = s % N_SLOTS
        return pltpu.async_copy(
            bufs.at[slot],
            out_hbm.at[pl.ds(base + s * BATCH, BATCH)],
            osems[slot],
        )

    gd = {}
    od = {}
    for s in range(LOOKAHEAD):
        gd[s] = start_gather(s)

    for s in range(SEQ_W):
        h = s + LOOKAHEAD
        if h < SEQ_W:
            prev = h - N_SLOTS
            if prev >= 0:
                od.pop(prev).wait()
            gd[h] = start_gather(h)

        gd.pop(s).wait()

        slot = s % N_SLOTS
        pe_regs = [pe_v[pl.ds(s * DIM + j * LANES, LANES)] for j in range(VPR)]

        def row_body(r, c, pe_regs=pe_regs, slot=slot):
            for j in range(VPR):
                v = bufs[slot, r, pl.ds(j * LANES, LANES)]
                bufs[slot, r, pl.ds(j * LANES, LANES)] = v * SCALE + pe_regs[j]
            return c

        lax.fori_loop(0, BATCH, row_body, 0, unroll=2)

        od[s] = start_out(s)

    for s in sorted(od):
        od.pop(s).wait()


@functools.cache
def _build_sc():
    mesh = plsc.VectorSubcoreMesh(
        core_axis_name="c", subcore_axis_name="s", num_cores=NC, num_subcores=NS
    )
    return pl.kernel(
        _sc_body,
        out_type=jax.ShapeDtypeStruct((ROWS, DIM), jnp.float32),
        mesh=mesh,
        scratch_types=[
            pltpu.VMEM((ROWS_W,), jnp.int32),
            pltpu.VMEM((SEQ_W * DIM,), jnp.float32),
            pltpu.VMEM((N_SLOTS, BATCH, DIM), jnp.float32),
            [pltpu.SemaphoreType.DMA] * N_SLOTS,
            [pltpu.SemaphoreType.DMA] * N_SLOTS,
        ],
        compiler_params=pltpu.CompilerParams(use_tc_tiling_on_sc=False),
    )


def kernel(source, word_table, pe):
    idx = source.reshape(ROWS)
    pe_flat = pe[:SEQ_LEN, 0, :].reshape(SEQ_LEN * DIM)
    packed = _build_sc()(idx, word_table, pe_flat)
    return packed.reshape(SEQ_LEN, BATCH, DIM)


# speedup vs baseline: 1.6594x; 1.3837x over previous
import functools
import math

import jax
import jax.numpy as jnp
from jax import lax
from jax.experimental import pallas as pl
from jax.experimental.pallas import tpu as pltpu
from jax.experimental.pallas import tpu_sc as plsc

SEQ_LEN = 2048
BATCH = 64
DIM = 64
NC = 2
NS = 16
NW = NC * NS
ROWS = SEQ_LEN * BATCH
ROWS_W = ROWS // NW
SEQ_W = SEQ_LEN // NW
CHUNK_S = 2
CHUNK_R = CHUNK_S * BATCH
N_CHUNKS = SEQ_W // CHUNK_S
SCALE = math.sqrt(DIM)
LANES = 16
VPR = DIM // LANES

N_SLOTS = 4
LOOKAHEAD = 2


@functools.cache
def _build_kernel():
    mesh = plsc.VectorSubcoreMesh(
        core_axis_name="c", subcore_axis_name="s", num_cores=NC, num_subcores=NS
    )
    return pl.kernel(
        _emb_body,
        out_type=jax.ShapeDtypeStruct((ROWS, 128), jnp.float32),
        mesh=mesh,
        scratch_types=[
            pltpu.VMEM((ROWS_W,), jnp.int32),
            pltpu.VMEM((SEQ_W * DIM,), jnp.float32),
            pltpu.VMEM((N_SLOTS, CHUNK_R, DIM), jnp.float32),
            [pltpu.SemaphoreType.DMA] * N_SLOTS,
            [pltpu.SemaphoreType.DMA] * N_SLOTS,
        ],
        compiler_params=pltpu.CompilerParams(use_tc_tiling_on_sc=False),
    )


def _emb_body(idx_hbm, table_hbm, pe_hbm, out_hbm, idx_v, pe_v, bufs, gsems, osems):
    wid = lax.axis_index("s") * NC + lax.axis_index("c")
    base = wid * ROWS_W

    pltpu.sync_copy(idx_hbm.at[pl.ds(base, ROWS_W)], idx_v)
    pltpu.sync_copy(pe_hbm.at[pl.ds(wid * SEQ_W * DIM, SEQ_W * DIM)], pe_v)

    def start_gather(g):
        slot = g % N_SLOTS
        idx_slice = idx_v.at[pl.ds(g * CHUNK_R, CHUNK_R)]
        return pltpu.async_copy(table_hbm.at[idx_slice], bufs.at[slot], gsems[slot])

    def start_out(g):
        slot = g % N_SLOTS
        return pltpu.async_copy(
            bufs.at[slot],
            out_hbm.at[pl.ds(base + g * CHUNK_R, CHUNK_R), pl.ds(0, DIM)],
            osems[slot],
        )

    gd = {}
    od = {}
    for g in range(LOOKAHEAD):
        gd[g] = start_gather(g)

    for g in range(N_CHUNKS):
        h = g + LOOKAHEAD
        if h < N_CHUNKS:
            prev = h - N_SLOTS
            if prev >= 0:
                od.pop(prev).wait()
            gd[h] = start_gather(h)

        gd.pop(g).wait()

        slot = g % N_SLOTS
        for sp in range(CHUNK_S):
            srow = g * CHUNK_S + sp
            pe_regs = [
                pe_v[pl.ds(srow * DIM + j * LANES, LANES)] for j in range(VPR)
            ]

            def row_body(r, c, pe_regs=pe_regs, sp=sp, slot=slot):
                row = sp * BATCH + r
                for j in range(VPR):
                    v = bufs[slot, row, pl.ds(j * LANES, LANES)]
                    bufs[slot, row, pl.ds(j * LANES, LANES)] = v * SCALE + pe_regs[j]
                return c

            lax.fori_loop(0, BATCH, row_body, 0, unroll=2)

        od[g] = start_out(g)

    for g in sorted(od):
        od.pop(g).wait()


def kernel(source, word_table, pe):
    idx = source.reshape(ROWS)
    pe_flat = pe[:SEQ_LEN, 0, :].reshape(SEQ_LEN * DIM)
    out = _build_kernel()(idx, word_table, pe_flat)
    return out.reshape(SEQ_LEN, BATCH, 128)[:, :, :DIM]


# speedup vs baseline: 1.6601x; 1.0004x over previous
import functools
import math

import jax
import jax.numpy as jnp
from jax import lax
from jax.experimental import pallas as pl
from jax.experimental.pallas import tpu as pltpu
from jax.experimental.pallas import tpu_sc as plsc

SEQ_LEN = 2048
BATCH = 64
DIM = 64
NC = 2
NS = 16
NW = NC * NS
ROWS = SEQ_LEN * BATCH
ROWS_W = ROWS // NW
SEQ_W = SEQ_LEN // NW
CHUNK_S = 2
CHUNK_R = CHUNK_S * BATCH
N_CHUNKS = SEQ_W // CHUNK_S
SCALE = math.sqrt(DIM)
LANES = 16
VPR = DIM // LANES

N_SLOTS = 6
LOOKAHEAD = 3


@functools.cache
def _build_kernel():
    mesh = plsc.VectorSubcoreMesh(
        core_axis_name="c", subcore_axis_name="s", num_cores=NC, num_subcores=NS
    )
    return pl.kernel(
        _emb_body,
        out_type=jax.ShapeDtypeStruct((ROWS, 128), jnp.float32),
        mesh=mesh,
        scratch_types=[
            pltpu.VMEM((ROWS_W,), jnp.int32),
            pltpu.VMEM((SEQ_W * DIM,), jnp.float32),
            pltpu.VMEM((N_SLOTS, CHUNK_R, DIM), jnp.float32),
            [pltpu.SemaphoreType.DMA] * N_SLOTS,
            [pltpu.SemaphoreType.DMA] * N_SLOTS,
        ],
        compiler_params=pltpu.CompilerParams(use_tc_tiling_on_sc=False),
    )


def _emb_body(idx_hbm, table_hbm, pe_hbm, out_hbm, idx_v, pe_v, bufs, gsems, osems):
    wid = lax.axis_index("s") * NC + lax.axis_index("c")
    base = wid * ROWS_W

    pltpu.sync_copy(idx_hbm.at[pl.ds(base, ROWS_W)], idx_v)
    pltpu.sync_copy(pe_hbm.at[pl.ds(wid * SEQ_W * DIM, SEQ_W * DIM)], pe_v)

    def start_gather(g):
        slot = g % N_SLOTS
        idx_slice = idx_v.at[pl.ds(g * CHUNK_R, CHUNK_R)]
        return pltpu.async_copy(table_hbm.at[idx_slice], bufs.at[slot], gsems[slot])

    def start_out(g):
        slot = g % N_SLOTS
        return pltpu.async_copy(
            bufs.at[slot],
            out_hbm.at[pl.ds(base + g * CHUNK_R, CHUNK_R), pl.ds(0, DIM)],
            osems[slot],
        )

    gd = {}
    od = {}
    for g in range(LOOKAHEAD):
        gd[g] = start_gather(g)

    for g in range(N_CHUNKS):
        h = g + LOOKAHEAD
        if h < N_CHUNKS:
            prev = h - N_SLOTS
            if prev >= 0:
                od.pop(prev).wait()
            gd[h] = start_gather(h)

        gd.pop(g).wait()

        slot = g % N_SLOTS
        for sp in range(CHUNK_S):
            srow = g * CHUNK_S + sp
            pe_regs = [
                pe_v[pl.ds(srow * DIM + j * LANES, LANES)] for j in range(VPR)
            ]

            def row_body(r, c, pe_regs=pe_regs, sp=sp, slot=slot):
                row = sp * BATCH + r
                for j in range(VPR):
                    v = bufs[slot, row, pl.ds(j * LANES, LANES)]
                    bufs[slot, row, pl.ds(j * LANES, LANES)] = v * SCALE + pe_regs[j]
                return c

            lax.fori_loop(0, BATCH, row_body, 0, unroll=4)

        od[g] = start_out(g)

    for g in sorted(od):
        od.pop(g).wait()


def kernel(source, word_table, pe):
    idx = source.reshape(ROWS)
    pe_flat = pe[:SEQ_LEN, 0, :].reshape(SEQ_LEN * DIM)
    out = _build_kernel()(idx, word_table, pe_flat)
    return out.reshape(SEQ_LEN, BATCH, 128)[:, :, :DIM]


# speedup vs baseline: 1.6608x; 1.0004x over previous
import functools
import math

import jax
import jax.numpy as jnp
from jax import lax
from jax.experimental import pallas as pl
from jax.experimental.pallas import tpu as pltpu
from jax.experimental.pallas import tpu_sc as plsc

SEQ_LEN = 2048
BATCH = 64
DIM = 64
NC = 2
NS = 16
NW = NC * NS
ROWS = SEQ_LEN * BATCH
ROWS_W = ROWS // NW
SEQ_W = SEQ_LEN // NW
CHUNK_S = 2
CHUNK_R = CHUNK_S * BATCH
N_CHUNKS = SEQ_W // CHUNK_S
SCALE = math.sqrt(DIM)
LANES = 16
VPR = DIM // LANES

N_SLOTS = 4
LOOKAHEAD = 2


@functools.cache
def _build_kernel():
    mesh = plsc.VectorSubcoreMesh(
        core_axis_name="c", subcore_axis_name="s", num_cores=NC, num_subcores=NS
    )
    return pl.kernel(
        _emb_body,
        out_type=jax.ShapeDtypeStruct((ROWS, 128), jnp.float32),
        mesh=mesh,
        scratch_types=[
            pltpu.VMEM((ROWS_W,), jnp.int32),
            pltpu.VMEM((SEQ_W * DIM,), jnp.float32),
            pltpu.VMEM((N_SLOTS, CHUNK_R, DIM), jnp.float32),
            [pltpu.SemaphoreType.DMA] * N_SLOTS,
            [pltpu.SemaphoreType.DMA] * N_SLOTS,
        ],
        compiler_params=pltpu.CompilerParams(use_tc_tiling_on_sc=False),
    )


def _emb_body(idx_hbm, table_hbm, pe_hbm, out_hbm, idx_v, pe_v, bufs, gsems, osems):
    wid = lax.axis_index("s") * NC + lax.axis_index("c")
    base = wid * ROWS_W

    pltpu.sync_copy(idx_hbm.at[pl.ds(base, ROWS_W)], idx_v)
    pltpu.sync_copy(pe_hbm.at[pl.ds(wid * SEQ_W * DIM, SEQ_W * DIM)], pe_v)

    def start_gather(g):
        slot = g % N_SLOTS
        idx_slice = idx_v.at[pl.ds(g * CHUNK_R, CHUNK_R)]
        return pltpu.async_copy(table_hbm.at[idx_slice], bufs.at[slot], gsems[slot])

    def start_out(g):
        slot = g % N_SLOTS
        return pltpu.async_copy(
            bufs.at[slot],
            out_hbm.at[pl.ds(base + g * CHUNK_R, CHUNK_R), pl.ds(0, DIM)],
            osems[slot],
        )

    gd = {}
    od = {}
    for g in range(LOOKAHEAD):
        gd[g] = start_gather(g)

    for g in range(N_CHUNKS):
        h = g + LOOKAHEAD
        if h < N_CHUNKS:
            prev = h - N_SLOTS
            if prev >= 0:
                od.pop(prev).wait()
            gd[h] = start_gather(h)

        gd.pop(g).wait()

        slot = g % N_SLOTS
        for sp in range(CHUNK_S):
            srow = g * CHUNK_S + sp
            pe_regs = [
                pe_v[pl.ds(srow * DIM + j * LANES, LANES)] for j in range(VPR)
            ]

            def row_body(r, c, pe_regs=pe_regs, sp=sp, slot=slot):
                row = sp * BATCH + r
                for j in range(VPR):
                    v = bufs[slot, row, pl.ds(j * LANES, LANES)]
                    bufs[slot, row, pl.ds(j * LANES, LANES)] = v * SCALE + pe_regs[j]
                return c

            lax.fori_loop(0, BATCH, row_body, 0, unroll=2)

        od[g] = start_out(g)

    for g in sorted(od):
        od.pop(g).wait()


def kernel(source, word_table, pe):
    idx = source.reshape(ROWS)
    pe_flat = pe[:SEQ_LEN, 0, :].reshape(SEQ_LEN * DIM)
    out = _build_kernel()(idx, word_table, pe_flat)
    return out.reshape(SEQ_LEN, BATCH, 128)[:, :, :DIM]
